# baseline (speedup 1.0000x reference)
import numpy as np
import jax
import jax.numpy as jnp
from jax import lax
from jax.experimental import pallas as pl
from jax.experimental.pallas import tpu as pltpu
from jax.experimental.pallas import tpu_sc as plsc

NSLICE = 6
H = W = 160
HW = H * W
SB = NSLICE * HW
SBP = 153856
TRASH = SB
NC, NS = 2, 16
ALPHAS = [float(a) for a in np.linspace(0.0, 1.0, NSLICE + 1, dtype=np.float32)]

CH = 32768
CHR = CH // 128


def _minmax_call(z, B, N):
    NCHUNK = N // CH

    def body(z_ref, lo_ref, hi_ref):
        c = pl.program_id(0)
        zv = z_ref[...]
        lo = jnp.min(zv, axis=(1, 2), keepdims=True)
        hi = jnp.max(zv, axis=(1, 2), keepdims=True)

        @pl.when(c == 0)
        def _():
            lo_ref[...] = lo
            hi_ref[...] = hi

        @pl.when(c != 0)
        def _():
            lo_ref[...] = jnp.minimum(lo_ref[...], lo)
            hi_ref[...] = jnp.maximum(hi_ref[...], hi)

    return pl.pallas_call(
        body,
        grid=(NCHUNK,),
        in_specs=[pl.BlockSpec((B, CHR, 128), lambda c: (0, c, 0))],
        out_specs=[pl.BlockSpec((B, 1, 1), lambda c: (0, 0, 0)),
                   pl.BlockSpec((B, 1, 1), lambda c: (0, 0, 0))],
        out_shape=[jax.ShapeDtypeStruct((B, 1, 1), jnp.float32),
                   jax.ShapeDtypeStruct((B, 1, 1), jnp.float32)],
    )(z.reshape(B, N // 128, 128))


NGROUP = 4


def _index_call(x, y, z, zlo, zhi, B, N, g):
    NCHUNK = N // CH
    BH = B // NGROUP
    chunk_spec = pl.BlockSpec((BH, CHR, 128), lambda c: (g, c, 0))
    scalar_spec = pl.BlockSpec((BH, 1, 1), lambda c: (g, 0, 0))

    def body(x_ref, y_ref, z_ref, lo_ref, hi_ref, idx_ref):
        xv = x_ref[...]
        yv = y_ref[...]
        zv = z_ref[...]
        lo = lo_ref[...]
        hi = hi_ref[...]
        gx = (xv - (-1.0)) / 2.000001 * (W - 1)
        gy = (yv - (-1.0)) / 2.000001 * (H - 1)
        valid = (gy >= 0.0) & (gy < H) & (gx >= 0.0) & (gx < W)
        iy = jnp.clip(gy.astype(jnp.int32), 0, H - 1)
        ix = jnp.clip(gx.astype(jnp.int32), 0, W - 1)
        flat = iy * W + ix
        s = jnp.zeros_like(flat)
        for j in range(1, NSLICE + 1):
            e = lo + (hi - lo) * ALPHAS[j]
            s += (zv >= e).astype(jnp.int32)
        idx_ref[...] = jnp.where(valid & (s < NSLICE), s * HW + flat, TRASH)

    out_spec = pl.BlockSpec((BH, CHR, 128), lambda c: (0, c, 0))
    return pl.pallas_call(
        body,
        grid=(NCHUNK,),
        in_specs=[chunk_spec, chunk_spec, chunk_spec, scalar_spec,
                  scalar_spec],
        out_specs=out_spec,
        out_shape=jax.ShapeDtypeStruct((BH, N // 128, 128), jnp.int32),
    )(x.reshape(B, N // 128, 128), y.reshape(B, N // 128, 128),
      z.reshape(B, N // 128, 128), zlo, zhi)


def _scatter_call(idx_flat, B, N):
    BPC = B // NC
    PPT = N // NS
    SHARE = SB // NS
    ZSHARE = SBP // NS

    mesh = plsc.VectorSubcoreMesh(core_axis_name="c", subcore_axis_name="s")

    def body(idx_hbm, out_hbm, idx_v, ones_v, zero_v, hist):
        cid = lax.axis_index("c")
        sid = lax.axis_index("s")

        def fill_ones(i, carry):
            for j in range(8):
                ones_v[pl.ds(i * 128 + j * 16, 16)] = jnp.full(
                    (16,), 1.0, jnp.float32)
            return carry

        def fill_zero(i, carry):
            for j in range(8):
                zero_v[pl.ds(i * 128 + j * 16, 16)] = jnp.zeros(
                    (16,), jnp.float32)
            return carry

        lax.fori_loop(0, PPT // 128, fill_ones, 0)
        lax.fori_loop(0, ZSHARE // 128, fill_zero, 0)

        for b in range(BPC):
            batch = cid * BPC + b
            pltpu.sync_copy(idx_hbm.at[pl.ds(batch * N + sid * PPT, PPT)],
                            idx_v)
            pltpu.sync_copy(zero_v, hist.at[pl.ds(sid * ZSHARE, ZSHARE)])
            plsc.subcore_barrier()
            pltpu.sync_copy(ones_v, hist.at[idx_v], add=True)
            plsc.subcore_barrier()
            pltpu.sync_copy(hist.at[pl.ds(sid * SHARE, SHARE)],
                            out_hbm.at[pl.ds(batch * SB + sid * SHARE,
                                             SHARE)])
            plsc.subcore_barrier()

    f = pl.kernel(
        body,
        out_type=jax.ShapeDtypeStruct((B * SB,), jnp.float32),
        mesh=mesh,
        scratch_types=[
            pltpu.VMEM((PPT,), jnp.int32),
            pltpu.VMEM((PPT,), jnp.float32),
            pltpu.VMEM((ZSHARE,), jnp.float32),
            pltpu.VMEM_SHARED((SBP,), jnp.float32),
        ],
    )
    return f(idx_flat)


def _normalize_call(counts, B):
    ROWS = HW // 128

    def body(c_ref, o_ref):
        bev = jnp.log1p(c_ref[...])
        bmin = jnp.min(bev)
        bmax = jnp.max(bev)
        o_ref[...] = (bev - bmin) / (bmax - bmin + 1e-6)

    return pl.pallas_call(
        body,
        grid=(B * NSLICE,),
        in_specs=[pl.BlockSpec((ROWS, 128), lambda i: (i, 0))],
        out_specs=pl.BlockSpec((ROWS, 128), lambda i: (i, 0)),
        out_shape=jax.ShapeDtypeStruct((B * NSLICE * ROWS, 128),
                                       jnp.float32),
    )(counts)


def kernel(xyz):
    B, N, _ = xyz.shape
    BH = B // NGROUP
    x = xyz[..., 0]
    y = xyz[..., 1]
    z = xyz[..., 2]
    zlo, zhi = _minmax_call(z, B, N)
    bevs = []
    for g in range(NGROUP):
        idx = _index_call(x, y, z, zlo, zhi, B, N, g)
        counts = _scatter_call(idx.reshape(BH * N), BH, N)
        bev = _normalize_call(
            counts.reshape(BH * NSLICE * (HW // 128), 128), BH)
        bevs.append(bev.reshape(BH, NSLICE, H, W))
    return jnp.concatenate(bevs, axis=0)


# speedup vs baseline: 1.0079x; 1.0079x over previous
import numpy as np
import jax
import jax.numpy as jnp
from jax import lax
from jax.experimental import pallas as pl
from jax.experimental.pallas import tpu as pltpu
from jax.experimental.pallas import tpu_sc as plsc

NSLICE = 6
H = W = 160
HW = H * W
SB = NSLICE * HW
SBP = 153856
TRASH = SB
NC, NS = 2, 16
ALPHAS = [float(a) for a in np.linspace(0.0, 1.0, NSLICE + 1, dtype=np.float32)]

CH = 32768
CHR = CH // 128


def _minmax_call(z, B, N):
    NCHUNK = N // CH

    def body(z_ref, lo_ref, hi_ref):
        c = pl.program_id(0)
        zv = z_ref[...]
        lo = jnp.min(zv, axis=(1, 2), keepdims=True)
        hi = jnp.max(zv, axis=(1, 2), keepdims=True)

        @pl.when(c == 0)
        def _():
            lo_ref[...] = lo
            hi_ref[...] = hi

        @pl.when(c != 0)
        def _():
            lo_ref[...] = jnp.minimum(lo_ref[...], lo)
            hi_ref[...] = jnp.maximum(hi_ref[...], hi)

    return pl.pallas_call(
        body,
        grid=(NCHUNK,),
        in_specs=[pl.BlockSpec((B, CHR, 128), lambda c: (0, c, 0))],
        out_specs=[pl.BlockSpec((B, 1, 1), lambda c: (0, 0, 0)),
                   pl.BlockSpec((B, 1, 1), lambda c: (0, 0, 0))],
        out_shape=[jax.ShapeDtypeStruct((B, 1, 1), jnp.float32),
                   jax.ShapeDtypeStruct((B, 1, 1), jnp.float32)],
    )(z.reshape(B, N // 128, 128))


NGROUP = 2


def _index_call(x, y, z, zlo, zhi, B, N, g):
    NCHUNK = N // CH
    BH = B // NGROUP
    chunk_spec = pl.BlockSpec((BH, CHR, 128), lambda c: (g, c, 0))
    scalar_spec = pl.BlockSpec((BH, 1, 1), lambda c: (g, 0, 0))

    def body(x_ref, y_ref, z_ref, lo_ref, hi_ref, idx_ref):
        xv = x_ref[...]
        yv = y_ref[...]
        zv = z_ref[...]
        lo = lo_ref[...]
        hi = hi_ref[...]
        gx = (xv - (-1.0)) / 2.000001 * (W - 1)
        gy = (yv - (-1.0)) / 2.000001 * (H - 1)
        valid = (gy >= 0.0) & (gy < H) & (gx >= 0.0) & (gx < W)
        iy = jnp.clip(gy.astype(jnp.int32), 0, H - 1)
        ix = jnp.clip(gx.astype(jnp.int32), 0, W - 1)
        flat = iy * W + ix
        s = jnp.zeros_like(flat)
        for j in range(1, NSLICE + 1):
            e = lo + (hi - lo) * ALPHAS[j]
            s += (zv >= e).astype(jnp.int32)
        idx_ref[...] = jnp.where(valid & (s < NSLICE), s * HW + flat, TRASH)

    out_spec = pl.BlockSpec((BH, CHR, 128), lambda c: (0, c, 0))
    return pl.pallas_call(
        body,
        grid=(NCHUNK,),
        in_specs=[chunk_spec, chunk_spec, chunk_spec, scalar_spec,
                  scalar_spec],
        out_specs=out_spec,
        out_shape=jax.ShapeDtypeStruct((BH, N // 128, 128), jnp.int32),
    )(x.reshape(B, N // 128, 128), y.reshape(B, N // 128, 128),
      z.reshape(B, N // 128, 128), zlo, zhi)


def _scatter_call(idx_flat, B, N):
    BPC = B // NC
    PPT = N // NS
    SHARE = SB // NS
    ZSHARE = SBP // NS

    mesh = plsc.VectorSubcoreMesh(core_axis_name="c", subcore_axis_name="s")

    def body(idx_hbm, out_hbm, idx_v, ones_v, zero_v, hist):
        cid = lax.axis_index("c")
        sid = lax.axis_index("s")

        def fill_ones(i, carry):
            for j in range(8):
                ones_v[pl.ds(i * 128 + j * 16, 16)] = jnp.full(
                    (16,), 1.0, jnp.float32)
            return carry

        def fill_zero(i, carry):
            for j in range(8):
                zero_v[pl.ds(i * 128 + j * 16, 16)] = jnp.zeros(
                    (16,), jnp.float32)
            return carry

        lax.fori_loop(0, PPT // 128, fill_ones, 0)
        lax.fori_loop(0, ZSHARE // 128, fill_zero, 0)

        for b in range(BPC):
            batch = cid * BPC + b
            pltpu.sync_copy(idx_hbm.at[pl.ds(batch * N + sid * PPT, PPT)],
                            idx_v)
            pltpu.sync_copy(zero_v, hist.at[pl.ds(sid * ZSHARE, ZSHARE)])
            plsc.subcore_barrier()
            pltpu.sync_copy(ones_v, hist.at[idx_v], add=True)
            plsc.subcore_barrier()
            pltpu.sync_copy(hist.at[pl.ds(sid * SHARE, SHARE)],
                            out_hbm.at[pl.ds(batch * SB + sid * SHARE,
                                             SHARE)])
            plsc.subcore_barrier()

    f = pl.kernel(
        body,
        out_type=jax.ShapeDtypeStruct((B * SB,), jnp.float32),
        mesh=mesh,
        scratch_types=[
            pltpu.VMEM((PPT,), jnp.int32),
            pltpu.VMEM((PPT,), jnp.float32),
            pltpu.VMEM((ZSHARE,), jnp.float32),
            pltpu.VMEM_SHARED((SBP,), jnp.float32),
        ],
    )
    return f(idx_flat)


def _normalize_call(counts, B):
    ROWS = HW // 128

    def body(c_ref, o_ref):
        bev = jnp.log1p(c_ref[...])
        bmin = jnp.min(bev)
        bmax = jnp.max(bev)
        o_ref[...] = (bev - bmin) / (bmax - bmin + 1e-6)

    return pl.pallas_call(
        body,
        grid=(B * NSLICE,),
        in_specs=[pl.BlockSpec((ROWS, 128), lambda i: (i, 0))],
        out_specs=pl.BlockSpec((ROWS, 128), lambda i: (i, 0)),
        out_shape=jax.ShapeDtypeStruct((B * NSLICE * ROWS, 128),
                                       jnp.float32),
    )(counts)


def kernel(xyz):
    B, N, _ = xyz.shape
    BH = B // NGROUP
    x = xyz[..., 0]
    y = xyz[..., 1]
    z = xyz[..., 2]
    zlo, zhi = _minmax_call(z, B, N)
    bevs = []
    for g in range(NGROUP):
        idx = _index_call(x, y, z, zlo, zhi, B, N, g)
        counts = _scatter_call(idx.reshape(BH * N), BH, N)
        bev = _normalize_call(
            counts.reshape(BH * NSLICE * (HW // 128), 128), BH)
        bevs.append(bev.reshape(BH, NSLICE, H, W))
    return jnp.concatenate(bevs, axis=0)
